# Initial kernel scaffold; baseline (speedup 1.0000x reference)
import jax
import jax.numpy as jnp
from jax.experimental import pallas as pl


def kernel(x, sample_map):
    raise NotImplementedError("write your pallas kernel here")


# speedup vs baseline: 41.4714x; 41.4714x over previous
import jax
import jax.numpy as jnp
from jax import lax
from jax.experimental import pallas as pl
from jax.experimental.pallas import tpu as pltpu
from jax.experimental.pallas import tpu_sc as plsc

_B, _C, _H, _W = 4, 16, 512, 1024
_F, _GRID = 80, 66
_K = _GRID * _GRID
_N = _F * _K
_CH = _B * _C
_NC, _NS, _L = 2, 16, 16
_NW = _NC * _NS
_CHUNK = 128
_CPW = -(-_N // (_NW * _CHUNK))
_NPAD = _NW * _CHUNK * _CPW


def _sc_body(table, sx, sy, out, sxv, syv, i00, i01, i10, i11, wxv, wyv,
             r00, r01, r10, r11, outv, sem):
    wid = lax.axis_index("s") * _NC + lax.axis_index("c")
    base = wid * (_CPW * _CHUNK)

    def chunk_body(g, carry):
        off = base + g * _CHUNK
        pltpu.sync_copy(sx.at[pl.ds(off, _CHUNK)], sxv)
        pltpu.sync_copy(sy.at[pl.ds(off, _CHUNK)], syv)
        for j in range(_CHUNK // _L):
            s = pl.ds(j * _L, _L)
            sxj = sxv[s]
            syj = syv[s]
            x0 = sxj.astype(jnp.int32)
            y0 = syj.astype(jnp.int32)
            wxv[s] = sxj - x0.astype(jnp.float32)
            wyv[s] = syj - y0.astype(jnp.float32)
            x0 = jnp.remainder(x0, _W)
            x1 = jnp.remainder(x0 + 1, _W)
            y0 = jnp.clip(y0, 0, _H - 1)
            y1 = jnp.minimum(y0 + 1, _H - 1)
            i00[s] = y0 * _W + x0
            i01[s] = y0 * _W + x1
            i10[s] = y1 * _W + x0
            i11[s] = y1 * _W + x1
        cp0 = pltpu.async_copy(table.at[i00], r00, sem)
        cp1 = pltpu.async_copy(table.at[i01], r01, sem)
        cp2 = pltpu.async_copy(table.at[i10], r10, sem)
        cp3 = pltpu.async_copy(table.at[i11], r11, sem)
        cp0.wait()
        cp1.wait()
        cp2.wait()
        cp3.wait()

        def grp_body(jg, c):
            s = pl.ds(jg * _L, _L)
            wxg = wxv[s]
            wyg = wyv[s]
            w11g = wxg * wyg
            w10g = wyg - w11g
            w01g = wxg - w11g
            w00g = 1.0 - wxg - wyg + w11g
            for l in range(_L):
                i = jg * _L + l
                w00, w01, w10, w11 = w00g[l], w01g[l], w10g[l], w11g[l]
                for cv in range(_CH // _L):
                    t = pl.ds(cv * _L, _L)
                    outv[i, t] = (r00[i, t] * w00 + r01[i, t] * w01
                                  + r10[i, t] * w10 + r11[i, t] * w11)
            return c

        lax.fori_loop(0, _CHUNK // _L, grp_body, 0)
        pltpu.sync_copy(outv, out.at[pl.ds(off, _CHUNK)])
        return carry

    lax.fori_loop(0, _CPW, chunk_body, 0)


def kernel(x, sample_map):
    table = x.transpose(2, 3, 0, 1).reshape(_H * _W, _CH)
    sx = sample_map[..., 0].reshape(-1)
    sy = sample_map[..., 1].reshape(-1)
    pad = _NPAD - _N
    sx = jnp.concatenate([sx, jnp.zeros((pad,), jnp.float32)])
    sy = jnp.concatenate([sy, jnp.zeros((pad,), jnp.float32)])
    mesh = plsc.VectorSubcoreMesh(core_axis_name="c", subcore_axis_name="s")
    out_flat = pl.kernel(
        _sc_body,
        out_type=jax.ShapeDtypeStruct((_NPAD, _CH), jnp.float32),
        mesh=mesh,
        compiler_params=pltpu.CompilerParams(use_tc_tiling_on_sc=False),
        scratch_types=[
            pltpu.VMEM((_CHUNK,), jnp.float32),
            pltpu.VMEM((_CHUNK,), jnp.float32),
            pltpu.VMEM((_CHUNK,), jnp.int32),
            pltpu.VMEM((_CHUNK,), jnp.int32),
            pltpu.VMEM((_CHUNK,), jnp.int32),
            pltpu.VMEM((_CHUNK,), jnp.int32),
            pltpu.VMEM((_CHUNK,), jnp.float32),
            pltpu.VMEM((_CHUNK,), jnp.float32),
            pltpu.VMEM((_CHUNK, _CH), jnp.float32),
            pltpu.VMEM((_CHUNK, _CH), jnp.float32),
            pltpu.VMEM((_CHUNK, _CH), jnp.float32),
            pltpu.VMEM((_CHUNK, _CH), jnp.float32),
            pltpu.VMEM((_CHUNK, _CH), jnp.float32),
            pltpu.SemaphoreType.DMA,
        ],
    )(table, sx, sy)
    out = out_flat[:_N].T.reshape(_B, _C, _F, _GRID, _GRID)
    return out


# speedup vs baseline: 49.4067x; 1.1913x over previous
import jax
import jax.numpy as jnp
from jax import lax
from jax.experimental import pallas as pl
from jax.experimental.pallas import tpu as pltpu
from jax.experimental.pallas import tpu_sc as plsc

_B, _C, _H, _W = 4, 16, 512, 1024
_F, _GRID = 80, 66
_K = _GRID * _GRID
_N = _F * _K
_CH = _B * _C
_NC, _NS, _L = 2, 16, 16
_NW = _NC * _NS
_CHUNK = 128
_CPW = -(-_N // (_NW * _CHUNK))
_SPAN = _CPW * _CHUNK
_NPAD = _NW * _SPAN


def _sc_body(table, sx, sy, out, sxa, sya, idx, wgt, rows, outv, gsem, osem):
    wid = lax.axis_index("s") * _NC + lax.axis_index("c")
    base = wid * _SPAN

    pltpu.sync_copy(sx.at[pl.ds(base, _SPAN)], sxa)
    pltpu.sync_copy(sy.at[pl.ds(base, _SPAN)], sya)

    def fire(g, p):
        for j in range(_CHUNK // _L):
            s = pl.ds(g * _CHUNK + j * _L, _L)
            d = pl.ds(j * _L, _L)
            sxj = sxa[s]
            syj = sya[s]
            x0 = sxj.astype(jnp.int32)
            y0 = syj.astype(jnp.int32)
            wgt[p][0][d] = sxj - x0.astype(jnp.float32)
            wgt[p][1][d] = syj - y0.astype(jnp.float32)
            x0 = jnp.remainder(x0, _W)
            x1 = jnp.remainder(x0 + 1, _W)
            y0 = jnp.clip(y0, 0, _H - 1)
            y1 = jnp.minimum(y0 + 1, _H - 1)
            r0 = y0 * _W
            r1 = y1 * _W
            idx[p][0][d] = r0 + x0
            idx[p][1][d] = r0 + x1
            idx[p][2][d] = r1 + x0
            idx[p][3][d] = r1 + x1
        for q in range(4):
            pltpu.async_copy(table.at[idx[p][q]], rows[p][q], gsem[p])

    fire(0, 0)

    def outer(gg, carry):
        for b in range(2):
            g = gg * 2 + b
            off = base + g * _CHUNK

            @pl.when(g < _CPW - 1)
            def _():
                fire(g + 1, 1 - b)

            for q in range(4):
                pltpu.make_async_copy(
                    table.at[pl.ds(0, _CHUNK)], rows[b][q], gsem[b]).wait()

            @pl.when(gg > 0)
            def _():
                pltpu.make_async_copy(
                    outv[b], out.at[pl.ds(0, _CHUNK)], osem[b]).wait()

            r00, r01, r10, r11 = rows[b]

            def grp_body(jg, c):
                s = pl.ds(jg * _L, _L)
                wxg = wgt[b][0][s]
                wyg = wgt[b][1][s]
                w11g = wxg * wyg
                w10g = wyg - w11g
                w01g = wxg - w11g
                w00g = 1.0 - wxg - wyg + w11g
                for l in range(_L):
                    i = jg * _L + l
                    w00, w01, w10, w11 = w00g[l], w01g[l], w10g[l], w11g[l]
                    for cv in range(_CH // _L):
                        t = pl.ds(cv * _L, _L)
                        outv[b][i, t] = (r00[i, t] * w00 + r01[i, t] * w01
                                         + r10[i, t] * w10 + r11[i, t] * w11)
                return c

            lax.fori_loop(0, _CHUNK // _L, grp_body, 0)
            pltpu.async_copy(outv[b], out.at[pl.ds(off, _CHUNK)], osem[b])
        return carry

    lax.fori_loop(0, _CPW // 2, outer, 0)
    for b in range(2):
        pltpu.make_async_copy(outv[b], out.at[pl.ds(0, _CHUNK)], osem[b]).wait()


def kernel(x, sample_map):
    table = x.transpose(2, 3, 0, 1).reshape(_H * _W, _CH)
    sx = sample_map[..., 0].reshape(-1)
    sy = sample_map[..., 1].reshape(-1)
    pad = _NPAD - _N
    sx = jnp.concatenate([sx, jnp.zeros((pad,), jnp.float32)])
    sy = jnp.concatenate([sy, jnp.zeros((pad,), jnp.float32)])
    mesh = plsc.VectorSubcoreMesh(core_axis_name="c", subcore_axis_name="s")
    out_flat = pl.kernel(
        _sc_body,
        out_type=jax.ShapeDtypeStruct((_NPAD, _CH), jnp.float32),
        mesh=mesh,
        compiler_params=pltpu.CompilerParams(use_tc_tiling_on_sc=False),
        scratch_types=[
            pltpu.VMEM((_SPAN,), jnp.float32),
            pltpu.VMEM((_SPAN,), jnp.float32),
            [[pltpu.VMEM((_CHUNK,), jnp.int32) for _ in range(4)]
             for _ in range(2)],
            [[pltpu.VMEM((_CHUNK,), jnp.float32) for _ in range(2)]
             for _ in range(2)],
            [[pltpu.VMEM((_CHUNK, _CH), jnp.float32) for _ in range(4)]
             for _ in range(2)],
            [pltpu.VMEM((_CHUNK, _CH), jnp.float32) for _ in range(2)],
            [pltpu.SemaphoreType.DMA for _ in range(2)],
            [pltpu.SemaphoreType.DMA for _ in range(2)],
        ],
    )(table, sx, sy)
    out = out_flat[:_N].T.reshape(_B, _C, _F, _GRID, _GRID)
    return out


# speedup vs baseline: 50.8673x; 1.0296x over previous
import jax
import jax.numpy as jnp
from jax import lax
from jax.experimental import pallas as pl
from jax.experimental.pallas import tpu as pltpu
from jax.experimental.pallas import tpu_sc as plsc

_B, _C, _H, _W = 4, 16, 512, 1024
_F, _GRID = 80, 66
_K = _GRID * _GRID
_N = _F * _K
_CH = _B * _C
_NC, _NS, _L = 2, 16, 16
_NW = _NC * _NS
_CHUNK = 96
_NCHUNK = _N // _CHUNK
_PAIRS = _NCHUNK // 2
_PAIRS_LO = _PAIRS // _NW
_PAIRS_EXTRA = _PAIRS % _NW
_MAXPAIRS = _PAIRS_LO + 1
_MAXSPAN = 2 * _MAXPAIRS * _CHUNK


def _sc_body(table, smf, out, sma, idx, wgt, rows, outt, gsem, osem):
    wid = lax.axis_index("s") * _NC + lax.axis_index("c")
    npairs = _PAIRS_LO + (wid < _PAIRS_EXTRA).astype(jnp.int32)
    start = 2 * (wid * _PAIRS_LO + jnp.minimum(wid, _PAIRS_EXTRA))
    base_s = start * _CHUNK

    win = jnp.minimum(2 * base_s, 2 * _N - 2 * _MAXSPAN)
    delta = 2 * base_s - win
    pltpu.sync_copy(smf.at[pl.ds(win, 2 * _MAXSPAN)], sma)

    iota = lax.iota(jnp.int32, _L)
    rowi = [cv * _L + iota for cv in range(_CH // _L)]

    def fire(g, p):
        for j in range(_CHUNK // _L):
            d = pl.ds(j * _L, _L)
            pos = delta + (g * _CHUNK + j * _L) * 2 + 2 * iota
            sxj = plsc.load_gather(sma, [pos])
            syj = plsc.load_gather(sma, [pos + 1])
            x0 = sxj.astype(jnp.int32)
            y0 = syj.astype(jnp.int32)
            wgt[p][0][d] = sxj - x0.astype(jnp.float32)
            wgt[p][1][d] = syj - y0.astype(jnp.float32)
            x0 = jnp.remainder(x0, _W)
            x1 = jnp.remainder(x0 + 1, _W)
            y0 = jnp.clip(y0, 0, _H - 1)
            y1 = jnp.minimum(y0 + 1, _H - 1)
            r0 = y0 * _W
            r1 = y1 * _W
            idx[p][0][d] = r0 + x0
            idx[p][1][d] = r0 + x1
            idx[p][2][d] = r1 + x0
            idx[p][3][d] = r1 + x1
        for q in range(4):
            pltpu.async_copy(table.at[idx[p][q]], rows[p][q], gsem[p])

    fire(0, 0)

    def outer(gg, carry):
        @pl.when(gg < npairs)
        def _():
            for b in range(2):
                g = gg * 2 + b

                if b == 0:
                    fire(g + 1, 1 - b)
                else:
                    @pl.when(gg + 1 < npairs)
                    def _():
                        fire(g + 1, 1 - b)

                for q in range(4):
                    pltpu.make_async_copy(
                        table.at[pl.ds(0, _CHUNK)], rows[b][q], gsem[b]).wait()

                @pl.when(gg > 0)
                def _():
                    pltpu.make_async_copy(
                        outt[b].at[:, pl.ds(0, _CHUNK)],
                        out.at[:, pl.ds(0, _CHUNK)], osem[b]).wait()

                r00, r01, r10, r11 = rows[b]

                def grp_body(jg, c):
                    s = pl.ds(jg * _L, _L)
                    wxg = wgt[b][0][s]
                    wyg = wgt[b][1][s]
                    w11g = wxg * wyg
                    w10g = wyg - w11g
                    w01g = wxg - w11g
                    w00g = 1.0 - wxg - wyg + w11g
                    for l in range(_L):
                        i = jg * _L + l
                        coli = jnp.full((_L,), i, jnp.int32)
                        w00, w01 = w00g[l], w01g[l]
                        w10, w11 = w10g[l], w11g[l]
                        for cv in range(_CH // _L):
                            t = pl.ds(cv * _L, _L)
                            acc = (r00[i, t] * w00 + r01[i, t] * w01
                                   + r10[i, t] * w10 + r11[i, t] * w11)
                            plsc.store_scatter(outt[b], [rowi[cv], coli], acc)
                    return c

                lax.fori_loop(0, _CHUNK // _L, grp_body, 0)
                off = (start + g) * _CHUNK
                pltpu.async_copy(outt[b].at[:, pl.ds(0, _CHUNK)],
                                 out.at[:, pl.ds(off, _CHUNK)], osem[b])
        return carry

    lax.fori_loop(0, _MAXPAIRS, outer, 0)
    for b in range(2):
        pltpu.make_async_copy(outt[b].at[:, pl.ds(0, _CHUNK)],
                              out.at[:, pl.ds(0, _CHUNK)], osem[b]).wait()


def kernel(x, sample_map):
    table = x.transpose(2, 3, 0, 1).reshape(_H * _W, _CH)
    smf = sample_map.reshape(2 * _N)
    mesh = plsc.VectorSubcoreMesh(core_axis_name="c", subcore_axis_name="s")
    out_t = pl.kernel(
        _sc_body,
        out_type=jax.ShapeDtypeStruct((_CH, _N), jnp.float32),
        mesh=mesh,
        compiler_params=pltpu.CompilerParams(use_tc_tiling_on_sc=False,
                                             needs_layout_passes=False),
        scratch_types=[
            pltpu.VMEM((2 * _MAXSPAN,), jnp.float32),
            [[pltpu.VMEM((_CHUNK,), jnp.int32) for _ in range(4)]
             for _ in range(2)],
            [[pltpu.VMEM((_CHUNK,), jnp.float32) for _ in range(2)]
             for _ in range(2)],
            [[pltpu.VMEM((_CHUNK, _CH), jnp.float32) for _ in range(4)]
             for _ in range(2)],
            [pltpu.VMEM((_CH, _CHUNK + 1), jnp.float32)
             for _ in range(2)],
            [pltpu.SemaphoreType.DMA for _ in range(2)],
            [pltpu.SemaphoreType.DMA for _ in range(2)],
        ],
    )(table, smf)
    return out_t.reshape(_B, _C, _F, _GRID, _GRID)
